# Initial kernel scaffold; baseline (speedup 1.0000x reference)
import jax
import jax.numpy as jnp
from jax.experimental import pallas as pl


def kernel(x, weight, hash_keys):
    raise NotImplementedError("write your pallas kernel here")


# speedup vs baseline: 4.5435x; 4.5435x over previous
import functools

import jax
import jax.numpy as jnp
from jax import lax
from jax.experimental import pallas as pl
from jax.experimental.pallas import tpu as pltpu
from jax.experimental.pallas import tpu_sc as plsc

VOCAB_SIZE = 1000000
DIM = 32
HASH_SIZE = int(VOCAB_SIZE * (1 - 0.95))

NC = 2
NS = 16
LANES = 16
NW = NC * NS

B = 16384 * 26
SPG = 8
CHUNK = 128
G = SPG * CHUNK
B_PER_W = B // NW
NGROUP = B_PER_W // G


def _sc_kernel(x_hbm, w_hbm, h_hbm, out_hbm, xg, hg, wg, rows, sem):
    wid = lax.axis_index("s") * NC + lax.axis_index("c")
    row0 = wid * (B_PER_W // CHUNK)
    base = wid * B_PER_W

    def body(g, _):
        pltpu.sync_copy(x_hbm.at[pl.ds(row0 + g * SPG, SPG)], xg)
        cps = [
            pltpu.async_copy(h_hbm.at[xg.at[j]], hg.at[j], sem)
            for j in range(SPG)
        ]
        for c in cps:
            c.wait()
        for j in range(SPG):
            for i in range(CHUNK // LANES):
                v = hg[j, pl.ds(i * LANES, LANES)]
                wg[j, pl.ds(i * LANES, LANES)] = lax.rem(
                    v, jnp.full((LANES,), HASH_SIZE, jnp.int32))
        cps = [
            pltpu.async_copy(
                w_hbm.at[wg.at[j]], rows.at[pl.ds(j * CHUNK, CHUNK)], sem)
            for j in range(SPG)
        ]
        for c in cps:
            c.wait()
        pltpu.sync_copy(rows, out_hbm.at[pl.ds(base + g * G, G)])
        return 0

    lax.fori_loop(0, NGROUP, body, 0)


@jax.jit
def _run(x2d, weight, hash_keys):
    mesh = plsc.VectorSubcoreMesh(core_axis_name="c", subcore_axis_name="s")
    out = pl.kernel(
        _sc_kernel,
        out_type=jax.ShapeDtypeStruct((B, DIM), jnp.float32),
        mesh=mesh,
        compiler_params=pltpu.CompilerParams(use_tc_tiling_on_sc=False),
        scratch_types=[
            pltpu.VMEM((SPG, CHUNK), jnp.int32),
            pltpu.VMEM((SPG, CHUNK), jnp.int32),
            pltpu.VMEM((SPG, CHUNK), jnp.int32),
            pltpu.VMEM((G, DIM), jnp.float32),
            pltpu.SemaphoreType.DMA,
        ],
    )(x2d, weight, hash_keys)
    return out


def kernel(x, weight, hash_keys):
    x2d = x.reshape(B // CHUNK, CHUNK)
    out = _run(x2d, weight, hash_keys)
    return out.reshape(x.shape[0], x.shape[1], DIM)


# speedup vs baseline: 6.4550x; 1.4207x over previous
import jax
import jax.numpy as jnp
from jax import lax
from jax.experimental import pallas as pl
from jax.experimental.pallas import tpu as pltpu
from jax.experimental.pallas import tpu_sc as plsc

VOCAB_SIZE = 1000000
DIM = 32
HASH_SIZE = int(VOCAB_SIZE * (1 - 0.95))

NC = 2
NS = 16
LANES = 16
NW = NC * NS

B = 16384 * 26
CHUNK = 128
SPG = 13
G = SPG * CHUNK
B_PER_W = B // NW
NGROUP = B_PER_W // G
NPAIR = NGROUP // 2


def _sc_kernel(x_hbm, w_hbm, h_hbm, out_hbm,
               xg0, xg1, hg0, hg1, wg0, wg1, rows0, rows1,
               sem_x, sem_h, sem_r, sem_o0, sem_o1):
    wid = lax.axis_index("s") * NC + lax.axis_index("c")
    base = wid * B_PER_W

    xg = (xg0, xg1)
    hg = (hg0, hg1)
    wg = (wg0, wg1)
    rows = (rows0, rows1)
    sem_o = (sem_o0, sem_o1)
    last = NGROUP - 1

    def x_copy(g, slot):
        g = jnp.minimum(g, last)
        return pltpu.make_async_copy(
            x_hbm.at[pl.ds(base + g * G, G)], xg[slot], sem_x)

    def fire_hash(slot):
        def f(j, _):
            pltpu.async_copy(
                h_hbm.at[xg[slot].at[pl.ds(j * CHUNK, CHUNK)]],
                hg[slot].at[pl.ds(j * CHUNK, CHUNK)], sem_h)
            return 0
        lax.fori_loop(0, SPG, f, 0)

    def wait_hash(slot):
        pltpu.make_async_copy(
            x_hbm.at[pl.ds(0, G)], hg[slot], sem_h).wait()

    def fire_rows(slot):
        def f(j, _):
            pltpu.async_copy(
                w_hbm.at[wg[slot].at[pl.ds(j * CHUNK, CHUNK)]],
                rows[slot].at[pl.ds(j * CHUNK, CHUNK)], sem_r)
            return 0
        lax.fori_loop(0, SPG, f, 0)

    def wait_rows(slot):
        pltpu.make_async_copy(
            out_hbm.at[pl.ds(0, G)], rows[slot], sem_r).wait()

    def out_copy(g, slot):
        return pltpu.make_async_copy(
            rows[slot], out_hbm.at[pl.ds(base + g * G, G)], sem_o[slot])

    def rem(slot):
        div = jnp.full((LANES,), HASH_SIZE, jnp.int32)
        def f(i, _):
            v = hg[slot][pl.ds(i * LANES, LANES)]
            wg[slot][pl.ds(i * LANES, LANES)] = lax.rem(v, div)
            return 0
        lax.fori_loop(0, G // LANES, f, 0)

    def half(k, g, slot):
        wait_hash(slot)
        x_copy(g + 1, 1 - slot).wait()
        fire_hash(1 - slot)
        x_copy(g + 2, slot).start()
        rem(slot)

        @pl.when(k > 0)
        def _():
            out_copy(g, slot).wait()

        fire_rows(slot)
        wait_rows(slot)
        out_copy(g, slot).start()

    pltpu.sync_copy(x_hbm.at[pl.ds(base, G)], xg0)
    fire_hash(0)
    x_copy(1, 1).start()

    def body(k, _):
        a = 2 * k
        half(k, a, 0)
        half(k, a + 1, 1)
        return 0

    lax.fori_loop(0, NPAIR, body, 0)

    x_copy(last, 1).wait()
    wait_hash(0)
    out_copy(last - 1, 0).wait()
    out_copy(last, 1).wait()


@jax.jit
def _run(x_flat, weight, hash_keys):
    mesh = plsc.VectorSubcoreMesh(core_axis_name="c", subcore_axis_name="s")
    out = pl.kernel(
        _sc_kernel,
        out_type=jax.ShapeDtypeStruct((B, DIM), jnp.float32),
        mesh=mesh,
        compiler_params=pltpu.CompilerParams(use_tc_tiling_on_sc=False),
        scratch_types=[
            pltpu.VMEM((G,), jnp.int32),
            pltpu.VMEM((G,), jnp.int32),
            pltpu.VMEM((G,), jnp.int32),
            pltpu.VMEM((G,), jnp.int32),
            pltpu.VMEM((G,), jnp.int32),
            pltpu.VMEM((G,), jnp.int32),
            pltpu.VMEM((G, DIM), jnp.float32),
            pltpu.VMEM((G, DIM), jnp.float32),
            pltpu.SemaphoreType.DMA,
            pltpu.SemaphoreType.DMA,
            pltpu.SemaphoreType.DMA,
            pltpu.SemaphoreType.DMA,
            pltpu.SemaphoreType.DMA,
        ],
    )(x_flat, weight, hash_keys)
    return out


def kernel(x, weight, hash_keys):
    out = _run(x.reshape(B), weight, hash_keys)
    return out.reshape(x.shape[0], x.shape[1], DIM)
